# baseline (speedup 1.0000x reference)
import functools

import jax
import jax.numpy as jnp
from jax import lax
from jax.experimental import pallas as pl
from jax.experimental.pallas import tpu as pltpu
from jax.experimental.pallas import tpu_sc as plsc

B = 16384
D = 64
NC = 2
NS = 16
NW = NC * NS
BPW = B // NW
CHUNK = 128
NCHUNK = BPW // CHUNK


def _neg_sqrt(x):
    i = lax.bitcast_convert_type(x, jnp.int32)
    y = lax.bitcast_convert_type(jnp.int32(0x5F3759DF) - (i >> 1), jnp.float32)
    for _ in range(3):
        y = y * (1.5 - 0.5 * x * y * y)
    return -(x * y)


def _sc_body(heads_h, rels_h, tails_h, etab_h, rtab_h, out_h,
             hidx_v, ridx_v, tidx_v, h_v, r_v, t_v, s_v, sem):
    wid = lax.axis_index("s") * NC + lax.axis_index("c")
    pltpu.sync_copy(heads_h.at[pl.ds(wid * NCHUNK, NCHUNK)], hidx_v)
    pltpu.sync_copy(rels_h.at[pl.ds(wid * NCHUNK, NCHUNK)], ridx_v)
    pltpu.sync_copy(tails_h.at[pl.ds(wid * NCHUNK, NCHUNK)], tidx_v)

    copies = []
    for j in range(NCHUNK):
        dst = pl.ds(j * CHUNK, CHUNK)
        copies.append(pltpu.async_copy(etab_h.at[hidx_v.at[j]], h_v.at[dst], sem))
        copies.append(pltpu.async_copy(rtab_h.at[ridx_v.at[j]], r_v.at[dst], sem))
        copies.append(pltpu.async_copy(etab_h.at[tidx_v.at[j]], t_v.at[dst], sem))
    for cp in copies:
        cp.wait()

    def group(g, carry):
        lane = lax.iota(jnp.int32, 16)
        vec = jnp.zeros((16,), jnp.float32)
        for k in range(16):
            ri = g * 16 + k
            acc = None
            for q in range(D // 16):
                sl = pl.ds(q * 16, 16)
                d = h_v[ri, sl] + r_v[ri, sl] - t_v[ri, sl]
                acc = d * d if acc is None else acc + d * d
            vec = jnp.where(lane == k, jnp.sum(acc), vec)
        s_v[pl.ds(g * 16, 16)] = _neg_sqrt(vec)
        return carry

    lax.fori_loop(0, BPW // 16, group, 0)
    pltpu.sync_copy(s_v, out_h.at[pl.ds(wid * BPW, BPW)])


@jax.jit
def _sc_call(heads2, rels2, tails2, entity_table, relation_table):
    mesh = plsc.VectorSubcoreMesh(core_axis_name="c", subcore_axis_name="s")
    run = functools.partial(
        pl.kernel,
        out_type=jax.ShapeDtypeStruct((B,), jnp.float32),
        mesh=mesh,
        compiler_params=pltpu.CompilerParams(
            needs_layout_passes=False, use_tc_tiling_on_sc=False),
        scratch_types=[
            pltpu.VMEM((NCHUNK, CHUNK), jnp.int32),
            pltpu.VMEM((NCHUNK, CHUNK), jnp.int32),
            pltpu.VMEM((NCHUNK, CHUNK), jnp.int32),
            pltpu.VMEM((BPW, D), jnp.float32),
            pltpu.VMEM((BPW, D), jnp.float32),
            pltpu.VMEM((BPW, D), jnp.float32),
            pltpu.VMEM((BPW,), jnp.float32),
            pltpu.SemaphoreType.DMA,
        ],
    )(_sc_body)
    return run(heads2, rels2, tails2, entity_table, relation_table)


def kernel(heads, relations, tails, entity_table, relation_table):
    h2 = heads.reshape(B // CHUNK, CHUNK)
    r2 = relations.reshape(B // CHUNK, CHUNK)
    t2 = tails.reshape(B // CHUNK, CHUNK)
    return _sc_call(h2, r2, t2, entity_table, relation_table)


# speedup vs baseline: 1.6444x; 1.6444x over previous
import functools

import jax
import jax.numpy as jnp
from jax import lax
from jax.experimental import pallas as pl
from jax.experimental.pallas import tpu as pltpu
from jax.experimental.pallas import tpu_sc as plsc

B = 16384
D = 64
NC = 2
NS = 16
NW = NC * NS
BPW = B // NW
CHUNK = 128
NCHUNK = BPW // CHUNK
CHUNK_BYTES = CHUNK * D * 4


def _neg_sqrt(x):
    i = lax.bitcast_convert_type(x, jnp.int32)
    y = lax.bitcast_convert_type(jnp.int32(0x5F3759DF) - (i >> 1), jnp.float32)
    for _ in range(3):
        y = y * (1.5 - 0.5 * x * y * y)
    return -(x * y)


def _sc_body(heads_h, rels_h, tails_h, etab_h, rtab_h, out_h,
             hidx_v, ridx_v, tidx_v, hbuf, rbuf, tbuf, s_v, sem):
    wid = lax.axis_index("s") * NC + lax.axis_index("c")
    base = wid * BPW
    pltpu.sync_copy(heads_h.at[pl.ds(base, BPW)], hidx_v)
    pltpu.sync_copy(rels_h.at[pl.ds(base, BPW)], ridx_v)
    pltpu.sync_copy(tails_h.at[pl.ds(base, BPW)], tidx_v)

    def issue(g, carry):
        i0 = g * 16
        hv = hidx_v[pl.ds(i0, 16)]
        rv = ridx_v[pl.ds(i0, 16)]
        tv = tidx_v[pl.ds(i0, 16)]
        row = i0 % CHUNK
        for k in range(16):
            pltpu.async_copy(
                etab_h.at[pl.ds(hv[k], 1)], hbuf.at[pl.ds(row + k, 1)], sem)
            pltpu.async_copy(
                rtab_h.at[pl.ds(rv[k], 1)], rbuf.at[pl.ds(row + k, 1)], sem)
            pltpu.async_copy(
                etab_h.at[pl.ds(tv[k], 1)], tbuf.at[pl.ds(row + k, 1)], sem)
        return carry

    def compute(g, carry):
        lane = lax.iota(jnp.int32, 16)
        vec = jnp.zeros((16,), jnp.float32)
        for k in range(16):
            ri = (g * 16 + k) % CHUNK
            acc = None
            for q in range(D // 16):
                sl = pl.ds(q * 16, 16)
                d = hbuf[ri, sl] + rbuf[ri, sl] - tbuf[ri, sl]
                acc = d * d if acc is None else acc + d * d
            vec = jnp.where(lane == k, jnp.sum(acc), vec)
        s_v[pl.ds(g * 16, 16)] = _neg_sqrt(vec)
        return carry

    for c in range(NCHUNK):
        gl, gh = c * (CHUNK // 16), (c + 1) * (CHUNK // 16)
        lax.fori_loop(gl, gh, issue, 0)
        pltpu.make_async_copy(etab_h.at[pl.ds(0, CHUNK)], hbuf, sem).wait()
        pltpu.make_async_copy(etab_h.at[pl.ds(0, CHUNK)], rbuf, sem).wait()
        pltpu.make_async_copy(etab_h.at[pl.ds(0, CHUNK)], tbuf, sem).wait()
        lax.fori_loop(gl, gh, compute, 0)

    pltpu.sync_copy(s_v, out_h.at[pl.ds(base, BPW)])


@jax.jit
def _sc_call(heads, relations, tails, entity_table, relation_table):
    mesh = plsc.VectorSubcoreMesh(core_axis_name="c", subcore_axis_name="s")
    run = functools.partial(
        pl.kernel,
        out_type=jax.ShapeDtypeStruct((B,), jnp.float32),
        mesh=mesh,
        compiler_params=pltpu.CompilerParams(needs_layout_passes=False),
        scratch_types=[
            pltpu.VMEM((BPW,), jnp.int32),
            pltpu.VMEM((BPW,), jnp.int32),
            pltpu.VMEM((BPW,), jnp.int32),
            pltpu.VMEM((CHUNK, D), jnp.float32),
            pltpu.VMEM((CHUNK, D), jnp.float32),
            pltpu.VMEM((CHUNK, D), jnp.float32),
            pltpu.VMEM((BPW,), jnp.float32),
            pltpu.SemaphoreType.DMA,
        ],
    )(_sc_body)
    return run(heads, relations, tails, entity_table, relation_table)


def kernel(heads, relations, tails, entity_table, relation_table):
    return _sc_call(heads, relations, tails, entity_table, relation_table)
